# baseline (speedup 1.0000x reference)
import jax
import jax.numpy as jnp
from jax import lax
from jax.experimental import pallas as pl
from jax.experimental.pallas import tpu as pltpu
from jax.experimental.pallas import tpu_sc as plsc

B = 4096
F = 26
V = 100000
D = 16
NPAIRS = F * (F - 1) // 2

_NC = 2
_NS = 16
_NW = _NC * _NS
_BPW = B // _NW
_VR = V // _NW
_HLEN = 1664
_HCAP2 = 1792
_OMAX = 96640
_HCAP = 512
_NCHK = (_HCAP + 16) // 64 + 1
_ACCR = 4352


def _phase1_body(tab_ref, tail_ref, idx_ref, out_ref,
                 win, idxf, offs, rowsf, rows2d, valb,
                 acc, semwin, semidx):
    cid = lax.axis_index("c")
    sid = lax.axis_index("s")
    wid = sid * _NC + cid
    lane = lax.iota(jnp.int32, 16)

    def zrow(r, carry):
        for j in range(8):
            valb[r, pl.ds(j * 16, 16)] = jnp.zeros(16, jnp.float32)
        return carry

    lax.fori_loop(0, 64, zrow, 0)
    zbase = sid * (_ACCR // _NS)
    for k in range(_ACCR // _NS // 16):
        pltpu.sync_copy(valb.at[pl.ds(0, 16)],
                        acc.at[pl.ds(zbase + k * 16, 16)])
    plsc.subcore_barrier()

    lo = wid * _VR
    hi = lo + _VR
    o = pl.multiple_of(jnp.minimum((lo >> 7) << 7, _OMAX), 128)
    mid = o + _HLEN
    has_tail = o == _OMAX

    def fire_half(f, h):
        pltpu.async_copy(tab_ref.at[f, :, pl.ds(o + h * _HLEN, _HLEN)],
                         win.at[h, :, pl.ds(0, _HLEN)], semwin)
        if h == 1:
            @pl.when(has_tail)
            def _():
                pltpu.async_copy(tail_ref.at[f],
                                 win.at[h, :, pl.ds(_HLEN, 128)], semwin)

    def wait_half(f, h):
        pltpu.make_async_copy(tab_ref.at[f, :, pl.ds(o + h * _HLEN, _HLEN)],
                              win.at[h, :, pl.ds(0, _HLEN)], semwin).wait()
        if h == 1:
            @pl.when(has_tail)
            def _():
                pltpu.make_async_copy(tail_ref.at[f],
                                      win.at[h, :, pl.ds(_HLEN, 128)],
                                      semwin).wait()

    pltpu.async_copy(idx_ref.at[0], idxf.at[0], semidx)
    fire_half(0, 0)
    fire_half(0, 1)

    def per_field(f, carry):
        db = f & 1
        pltpu.make_async_copy(idx_ref.at[f], idxf.at[db], semidx).wait()

        @pl.when(f + 1 < F)
        def _():
            pltpu.async_copy(idx_ref.at[f + 1], idxf.at[1 - db], semidx)

        def filt(kk, poss):
            p0, p1 = poss
            vv = idxf[db, kk >> 3, pl.ds((kk & 7) * 16, 16)]
            rr = kk * 16 + lane
            m0 = (vv >= lo) & (vv < mid)
            m1 = (vv >= mid) & (vv < hi)
            plsc.store_compressed(offs.at[0, pl.ds(p0, 16)], vv - o, mask=m0)
            plsc.store_compressed(rowsf.at[0, pl.ds(p0, 16)], rr, mask=m0)
            plsc.store_compressed(offs.at[1, pl.ds(p1, 16)], vv - mid,
                                  mask=m1)
            plsc.store_compressed(rowsf.at[1, pl.ds(p1, 16)], rr, mask=m1)
            c0 = plsc.all_reduce_population_count(m0)[0]
            c1 = plsc.all_reduce_population_count(m1)[0]
            return p0 + c0, p1 + c1

        p0, p1 = lax.fori_loop(0, B // 16, filt, (0, 0))
        for hh in range(2):
            ph = p0 if hh == 0 else p1
            offs[hh, pl.ds(ph, 16)] = jnp.zeros(16, jnp.int32)
            rowsf[hh, pl.ds(ph, 16)] = jnp.full((16,), B, jnp.int32)

        for h in range(2):
            pos = p0 if h == 0 else p1
            wait_half(f, h)
            ngrp = (pos + 15) >> 4
            nchunk = (ngrp + 3) >> 2

            def chunk(c, carry2):
                @pl.when(c < nchunk)
                def _():
                    for gg in range(4):
                        g = c * 4 + gg

                        @pl.when(g < ngrp)
                        def _():
                            o16 = offs[h, pl.ds(g * 16, 16)]
                            r16 = rowsf[h, pl.ds(g * 16, 16)]
                            rows2d[c, pl.ds(gg * 16, 16)] = r16
                            for i in range(16):
                                col = plsc.load_gather(
                                    win.at[h],
                                    [lane,
                                     jnp.full((16,), o16[i], jnp.int32)])
                                valb[gg * 16 + i, pl.ds(0, 16)] = col
                                valb[gg * 16 + i, pl.ds(16, 16)] = col * col

                        @pl.when(g >= ngrp)
                        def _():
                            rows2d[c, pl.ds(gg * 16, 16)] = jnp.full(
                                (16,), B, jnp.int32)
                            for i in range(16):
                                valb[gg * 16 + i, pl.ds(0, 16)] = jnp.zeros(
                                    16, jnp.float32)
                                valb[gg * 16 + i, pl.ds(16, 16)] = jnp.zeros(
                                    16, jnp.float32)

                    pltpu.sync_copy(valb.at[pl.ds(0, 64)],
                                    acc.at[rows2d.at[c]], add=True)
                return carry2

            lax.fori_loop(0, _NCHK, chunk, 0)

            @pl.when(f + 1 < F)
            def _():
                fire_half(f + 1, h)

        return carry

    lax.fori_loop(0, F, per_field, 0)
    plsc.subcore_barrier()
    ebase = sid * 256
    pltpu.sync_copy(acc.at[pl.ds(ebase, 256)],
                    out_ref.at[cid, pl.ds(ebase, 256)])


def _phase2_body(p_ref, aux_ref, out_ref, p0, p1, ov, aux_v):
    wid = lax.axis_index("s") * _NC + lax.axis_index("c")
    base = wid * _BPW
    pltpu.sync_copy(p_ref.at[0, pl.ds(base, _BPW)], p0)
    pltpu.sync_copy(p_ref.at[1, pl.ds(base, _BPW)], p1)
    pltpu.sync_copy(aux_ref, aux_v)
    wv = aux_v[pl.ds(0, D)] * (1.0 / (2.0 * NPAIRS))
    bv = aux_v[pl.ds(D, 16)]
    lane = lax.iota(jnp.int32, 16)
    for g in range(_BPW // 16):
        def body(j, acc):
            r = g * 16 + j
            s = p0[r, pl.ds(0, 16)] + p1[r, pl.ds(0, 16)]
            q = p0[r, pl.ds(16, 16)] + p1[r, pl.ds(16, 16)]
            x = (s * s - q) * wv
            z = jnp.sum(x)
            return jnp.where(lane == j, z, acc)

        acc = lax.fori_loop(0, 16, body, jnp.zeros(16, jnp.float32))
        ov[pl.ds(g * 16, 16)] = 1.0 / (1.0 + jnp.exp(-(acc + bv)))
    pltpu.sync_copy(ov, out_ref.at[pl.ds(wid * _BPW, _BPW)])


def kernel(dense_inputs, sparse_inputs, tables, W, b):
    del dense_inputs
    tab = jnp.transpose(tables, (0, 2, 1))
    tail = jnp.pad(tab[:, :, (V // 128) * 128:], ((0, 0), (0, 0), (0, 96)))
    idxT = sparse_inputs.T.reshape(F, B // 128, 128)
    aux = jnp.concatenate([W.reshape(D), jnp.broadcast_to(b, (16,))]
                          ).astype(jnp.float32)
    mesh = plsc.VectorSubcoreMesh(core_axis_name="c", subcore_axis_name="s")
    partials = pl.kernel(
        _phase1_body,
        mesh=mesh,
        compiler_params=pltpu.CompilerParams(
            needs_layout_passes=False, use_tc_tiling_on_sc=True),
        out_type=jax.ShapeDtypeStruct((_NC, B, 128), jnp.float32),
        scratch_types=[
            pltpu.VMEM((2, D, _HCAP2), jnp.float32),
            pltpu.VMEM((2, B // 128, 128), jnp.int32),
            pltpu.VMEM((2, _HCAP + 16), jnp.int32),
            pltpu.VMEM((2, _HCAP + 16), jnp.int32),
            pltpu.VMEM((_NCHK, 64), jnp.int32),
            pltpu.VMEM((64, 128), jnp.float32),
            pltpu.VMEM_SHARED((_ACCR, 128), jnp.float32),
            pltpu.SemaphoreType.DMA,
            pltpu.SemaphoreType.DMA,
        ],
    )(tab, tail, idxT)
    out = pl.kernel(
        _phase2_body,
        mesh=mesh,
        compiler_params=pltpu.CompilerParams(
            needs_layout_passes=False, use_tc_tiling_on_sc=True),
        out_type=jax.ShapeDtypeStruct((B,), jnp.float32),
        scratch_types=[
            pltpu.VMEM((_BPW, 128), jnp.float32),
            pltpu.VMEM((_BPW, 128), jnp.float32),
            pltpu.VMEM((_BPW,), jnp.float32),
            pltpu.VMEM((2 * 16,), jnp.float32),
        ],
    )(partials, aux)
    return out.reshape(B, 1)


# speedup vs baseline: 1.0035x; 1.0035x over previous
import jax
import jax.numpy as jnp
from jax import lax
from jax.experimental import pallas as pl
from jax.experimental.pallas import tpu as pltpu
from jax.experimental.pallas import tpu_sc as plsc

B = 4096
F = 26
V = 100000
D = 16
NPAIRS = F * (F - 1) // 2

_NC = 2
_NS = 16
_NW = _NC * _NS
_BPW = B // _NW
_VR = V // _NW
_HLEN = 1664
_HCAP2 = 1792
_OMAX = 96640
_HCAP = 512
_NCHK = (_HCAP + 16) // 32 + 1
_ACCR = 4352


def _phase1_body(tab_ref, tail_ref, idx_ref, out_ref,
                 win, idxf, offs, rowsf, rows2d, valb,
                 acc, semwin, semidx):
    cid = lax.axis_index("c")
    sid = lax.axis_index("s")
    wid = sid * _NC + cid
    lane = lax.iota(jnp.int32, 16)

    def zrow(r, carry):
        for j in range(8):
            valb[r, pl.ds(j * 16, 16)] = jnp.zeros(16, jnp.float32)
        return carry

    lax.fori_loop(0, 32, zrow, 0)
    zbase = sid * (_ACCR // _NS)
    for k in range(_ACCR // _NS // 16):
        pltpu.sync_copy(valb.at[pl.ds(0, 16)],
                        acc.at[pl.ds(zbase + k * 16, 16)])
    plsc.subcore_barrier()

    lo = wid * _VR
    hi = lo + _VR
    o = pl.multiple_of(jnp.minimum((lo >> 7) << 7, _OMAX), 128)
    mid = o + _HLEN
    has_tail = o == _OMAX

    def fire_half(f, h):
        pltpu.async_copy(tab_ref.at[f, :, pl.ds(o + h * _HLEN, _HLEN)],
                         win.at[h, :, pl.ds(0, _HLEN)], semwin)
        if h == 1:
            @pl.when(has_tail)
            def _():
                pltpu.async_copy(tail_ref.at[f],
                                 win.at[h, :, pl.ds(_HLEN, 128)], semwin)

    def wait_half(f, h):
        pltpu.make_async_copy(tab_ref.at[f, :, pl.ds(o + h * _HLEN, _HLEN)],
                              win.at[h, :, pl.ds(0, _HLEN)], semwin).wait()
        if h == 1:
            @pl.when(has_tail)
            def _():
                pltpu.make_async_copy(tail_ref.at[f],
                                      win.at[h, :, pl.ds(_HLEN, 128)],
                                      semwin).wait()

    pltpu.async_copy(idx_ref.at[0], idxf.at[0], semidx)
    fire_half(0, 0)
    fire_half(0, 1)

    def per_field(f, carry):
        db = f & 1
        pltpu.make_async_copy(idx_ref.at[f], idxf.at[db], semidx).wait()

        @pl.when(f + 1 < F)
        def _():
            pltpu.async_copy(idx_ref.at[f + 1], idxf.at[1 - db], semidx)

        def filt(kk, poss):
            p0, p1 = poss
            vv = idxf[db, kk >> 3, pl.ds((kk & 7) * 16, 16)]
            rr = kk * 16 + lane
            m0 = (vv >= lo) & (vv < mid)
            m1 = (vv >= mid) & (vv < hi)
            plsc.store_compressed(offs.at[0, pl.ds(p0, 16)], vv - o, mask=m0)
            plsc.store_compressed(rowsf.at[0, pl.ds(p0, 16)], rr, mask=m0)
            plsc.store_compressed(offs.at[1, pl.ds(p1, 16)], vv - mid,
                                  mask=m1)
            plsc.store_compressed(rowsf.at[1, pl.ds(p1, 16)], rr, mask=m1)
            c0 = plsc.all_reduce_population_count(m0)[0]
            c1 = plsc.all_reduce_population_count(m1)[0]
            return p0 + c0, p1 + c1

        p0, p1 = lax.fori_loop(0, B // 16, filt, (0, 0))
        for hh in range(2):
            ph = p0 if hh == 0 else p1
            offs[hh, pl.ds(ph, 16)] = jnp.zeros(16, jnp.int32)
            rowsf[hh, pl.ds(ph, 16)] = jnp.full((16,), B, jnp.int32)

        for h in range(2):
            pos = p0 if h == 0 else p1
            wait_half(f, h)
            ngrp = (pos + 15) >> 4
            nchunk = (ngrp + 1) >> 1

            def chunk(c, carry2):
                @pl.when(c < nchunk)
                def _():
                    for gg in range(2):
                        g = c * 2 + gg

                        @pl.when(g < ngrp)
                        def _():
                            o16 = offs[h, pl.ds(g * 16, 16)]
                            r16 = rowsf[h, pl.ds(g * 16, 16)]
                            rows2d[c, pl.ds(gg * 16, 16)] = r16
                            for i in range(16):
                                col = plsc.load_gather(
                                    win.at[h],
                                    [lane,
                                     jnp.full((16,), o16[i], jnp.int32)])
                                valb[gg * 16 + i, pl.ds(0, 16)] = col
                                valb[gg * 16 + i, pl.ds(16, 16)] = col * col

                        @pl.when(g >= ngrp)
                        def _():
                            rows2d[c, pl.ds(gg * 16, 16)] = jnp.full(
                                (16,), B, jnp.int32)
                            for i in range(16):
                                valb[gg * 16 + i, pl.ds(0, 16)] = jnp.zeros(
                                    16, jnp.float32)
                                valb[gg * 16 + i, pl.ds(16, 16)] = jnp.zeros(
                                    16, jnp.float32)

                    pltpu.sync_copy(valb.at[pl.ds(0, 32)],
                                    acc.at[rows2d.at[c]], add=True)
                return carry2

            lax.fori_loop(0, _NCHK, chunk, 0)

            @pl.when(f + 1 < F)
            def _():
                fire_half(f + 1, h)

        return carry

    lax.fori_loop(0, F, per_field, 0)
    plsc.subcore_barrier()
    ebase = sid * 256
    pltpu.sync_copy(acc.at[pl.ds(ebase, 256)],
                    out_ref.at[cid, pl.ds(ebase, 256)])


def _phase2_body(p_ref, aux_ref, out_ref, p0, p1, ov, aux_v):
    wid = lax.axis_index("s") * _NC + lax.axis_index("c")
    base = wid * _BPW
    pltpu.sync_copy(p_ref.at[0, pl.ds(base, _BPW)], p0)
    pltpu.sync_copy(p_ref.at[1, pl.ds(base, _BPW)], p1)
    pltpu.sync_copy(aux_ref, aux_v)
    wv = aux_v[pl.ds(0, D)] * (1.0 / (2.0 * NPAIRS))
    bv = aux_v[pl.ds(D, 16)]
    lane = lax.iota(jnp.int32, 16)
    for g in range(_BPW // 16):
        def body(j, acc):
            r = g * 16 + j
            s = p0[r, pl.ds(0, 16)] + p1[r, pl.ds(0, 16)]
            q = p0[r, pl.ds(16, 16)] + p1[r, pl.ds(16, 16)]
            x = (s * s - q) * wv
            z = jnp.sum(x)
            return jnp.where(lane == j, z, acc)

        acc = lax.fori_loop(0, 16, body, jnp.zeros(16, jnp.float32))
        ov[pl.ds(g * 16, 16)] = 1.0 / (1.0 + jnp.exp(-(acc + bv)))
    pltpu.sync_copy(ov, out_ref.at[pl.ds(wid * _BPW, _BPW)])


def kernel(dense_inputs, sparse_inputs, tables, W, b):
    del dense_inputs
    tab = jnp.transpose(tables, (0, 2, 1))
    tail = jnp.pad(tab[:, :, (V // 128) * 128:], ((0, 0), (0, 0), (0, 96)))
    idxT = sparse_inputs.T.reshape(F, B // 128, 128)
    aux = jnp.concatenate([W.reshape(D), jnp.broadcast_to(b, (16,))]
                          ).astype(jnp.float32)
    mesh = plsc.VectorSubcoreMesh(core_axis_name="c", subcore_axis_name="s")
    partials = pl.kernel(
        _phase1_body,
        mesh=mesh,
        compiler_params=pltpu.CompilerParams(
            needs_layout_passes=False, use_tc_tiling_on_sc=True),
        out_type=jax.ShapeDtypeStruct((_NC, B, 128), jnp.float32),
        scratch_types=[
            pltpu.VMEM((2, D, _HCAP2), jnp.float32),
            pltpu.VMEM((2, B // 128, 128), jnp.int32),
            pltpu.VMEM((2, _HCAP + 16), jnp.int32),
            pltpu.VMEM((2, _HCAP + 16), jnp.int32),
            pltpu.VMEM((_NCHK, 32), jnp.int32),
            pltpu.VMEM((32, 128), jnp.float32),
            pltpu.VMEM_SHARED((_ACCR, 128), jnp.float32),
            pltpu.SemaphoreType.DMA,
            pltpu.SemaphoreType.DMA,
        ],
    )(tab, tail, idxT)
    out = pl.kernel(
        _phase2_body,
        mesh=mesh,
        compiler_params=pltpu.CompilerParams(
            needs_layout_passes=False, use_tc_tiling_on_sc=True),
        out_type=jax.ShapeDtypeStruct((B,), jnp.float32),
        scratch_types=[
            pltpu.VMEM((_BPW, 128), jnp.float32),
            pltpu.VMEM((_BPW, 128), jnp.float32),
            pltpu.VMEM((_BPW,), jnp.float32),
            pltpu.VMEM((2 * 16,), jnp.float32),
        ],
    )(partials, aux)
    return out.reshape(B, 1)


# speedup vs baseline: 1.0501x; 1.0464x over previous
import jax
import jax.numpy as jnp
from jax import lax
from jax.experimental import pallas as pl
from jax.experimental.pallas import tpu as pltpu
from jax.experimental.pallas import tpu_sc as plsc

B = 4096
F = 26
V = 100000
D = 16
NPAIRS = F * (F - 1) // 2

_NC = 2
_NS = 16
_NW = _NC * _NS
_BPW = B // _NW
_VR = V // _NW
_HLEN = 1664
_HCAP2 = 1792
_OMAX = 96640
_HCAP = 512
_NCHK = (_HCAP + 16) // 32 + 1
_ACCR = 4352


def _phase1_body(tab_ref, tail_ref, idx_ref, out_ref,
                 win, idxf, offs, rowsf, rows2d, valb,
                 acc, semwin, semidx, semsc):
    cid = lax.axis_index("c")
    sid = lax.axis_index("s")
    wid = sid * _NC + cid
    lane = lax.iota(jnp.int32, 16)

    def zrow(r, carry):
        for a in range(2):
            for j in range(8):
                valb[a, r, pl.ds(j * 16, 16)] = jnp.zeros(16, jnp.float32)
        return carry

    lax.fori_loop(0, 32, zrow, 0)
    zbase = sid * (_ACCR // _NS)
    for k in range(_ACCR // _NS // 16):
        pltpu.sync_copy(valb.at[0, pl.ds(0, 16)],
                        acc.at[pl.ds(zbase + k * 16, 16)])
    plsc.subcore_barrier()

    lo = wid * _VR
    hi = lo + _VR
    o = pl.multiple_of(jnp.minimum((lo >> 7) << 7, _OMAX), 128)
    mid = o + _HLEN
    has_tail = o == _OMAX

    def fire_half(f, h):
        pltpu.async_copy(tab_ref.at[f, :, pl.ds(o + h * _HLEN, _HLEN)],
                         win.at[h, :, pl.ds(0, _HLEN)], semwin)
        if h == 1:
            @pl.when(has_tail)
            def _():
                pltpu.async_copy(tail_ref.at[f],
                                 win.at[h, :, pl.ds(_HLEN, 128)], semwin)

    def wait_half(f, h):
        pltpu.make_async_copy(tab_ref.at[f, :, pl.ds(o + h * _HLEN, _HLEN)],
                              win.at[h, :, pl.ds(0, _HLEN)], semwin).wait()
        if h == 1:
            @pl.when(has_tail)
            def _():
                pltpu.make_async_copy(tail_ref.at[f],
                                      win.at[h, :, pl.ds(_HLEN, 128)],
                                      semwin).wait()

    pltpu.async_copy(idx_ref.at[0], idxf.at[0], semidx)
    fire_half(0, 0)
    fire_half(0, 1)

    def per_field(f, carry):
        db = f & 1
        pltpu.make_async_copy(idx_ref.at[f], idxf.at[db], semidx).wait()

        @pl.when(f + 1 < F)
        def _():
            pltpu.async_copy(idx_ref.at[f + 1], idxf.at[1 - db], semidx)

        def filt(kk, poss):
            p0, p1 = poss
            vv = idxf[db, kk >> 3, pl.ds((kk & 7) * 16, 16)]
            rr = kk * 16 + lane
            m0 = (vv >= lo) & (vv < mid)
            m1 = (vv >= mid) & (vv < hi)
            plsc.store_compressed(offs.at[0, pl.ds(p0, 16)], vv - o, mask=m0)
            plsc.store_compressed(rowsf.at[0, pl.ds(p0, 16)], rr, mask=m0)
            plsc.store_compressed(offs.at[1, pl.ds(p1, 16)], vv - mid,
                                  mask=m1)
            plsc.store_compressed(rowsf.at[1, pl.ds(p1, 16)], rr, mask=m1)
            c0 = plsc.all_reduce_population_count(m0)[0]
            c1 = plsc.all_reduce_population_count(m1)[0]
            return p0 + c0, p1 + c1

        p0, p1 = lax.fori_loop(0, B // 16, filt, (0, 0))
        for hh in range(2):
            ph = p0 if hh == 0 else p1
            offs[hh, pl.ds(ph, 16)] = jnp.zeros(16, jnp.int32)
            rowsf[hh, pl.ds(ph, 16)] = jnp.full((16,), B, jnp.int32)

        for h in range(2):
            pos = p0 if h == 0 else p1
            wait_half(f, h)
            ngrp = (pos + 15) >> 4
            nchunk = (ngrp + 1) >> 1

            def build(c, a):
                for gg in range(2):
                    g = c * 2 + gg

                    @pl.when(g < ngrp)
                    def _():
                        o16 = offs[h, pl.ds(g * 16, 16)]
                        r16 = rowsf[h, pl.ds(g * 16, 16)]
                        rows2d[c, pl.ds(gg * 16, 16)] = r16
                        for i in range(16):
                            col = plsc.load_gather(
                                win.at[h],
                                [lane,
                                 jnp.full((16,), o16[i], jnp.int32)])
                            valb[a, gg * 16 + i, pl.ds(0, 16)] = col
                            valb[a, gg * 16 + i, pl.ds(16, 16)] = col * col

                    @pl.when(g >= ngrp)
                    def _():
                        rows2d[c, pl.ds(gg * 16, 16)] = jnp.full(
                            (16,), B, jnp.int32)
                        for i in range(16):
                            valb[a, gg * 16 + i, pl.ds(0, 16)] = jnp.zeros(
                                16, jnp.float32)
                            valb[a, gg * 16 + i, pl.ds(16, 16)] = jnp.zeros(
                                16, jnp.float32)

            def chunkpair(c2, carry2):
                ca = c2 * 2
                cb = ca + 1
                da = pltpu.make_async_copy(valb.at[0], acc.at[rows2d.at[ca]],
                                           semsc)
                db = pltpu.make_async_copy(valb.at[1], acc.at[rows2d.at[cb]],
                                           semsc)

                @pl.when(ca < nchunk)
                def _():
                    build(ca, 0)
                    da.start(add=True)

                @pl.when(cb < nchunk)
                def _():
                    build(cb, 1)
                    db.start(add=True)

                @pl.when(ca < nchunk)
                def _():
                    da.wait()

                @pl.when(cb < nchunk)
                def _():
                    db.wait()

                return carry2

            lax.fori_loop(0, (_NCHK + 1) // 2, chunkpair, 0)

            @pl.when(f + 1 < F)
            def _():
                fire_half(f + 1, h)

        return carry

    lax.fori_loop(0, F, per_field, 0)
    plsc.subcore_barrier()
    ebase = sid * 256
    pltpu.sync_copy(acc.at[pl.ds(ebase, 256)],
                    out_ref.at[cid, pl.ds(ebase, 256)])


def _phase2_body(p_ref, aux_ref, out_ref, p0, p1, ov, aux_v):
    wid = lax.axis_index("s") * _NC + lax.axis_index("c")
    base = wid * _BPW
    pltpu.sync_copy(p_ref.at[0, pl.ds(base, _BPW)], p0)
    pltpu.sync_copy(p_ref.at[1, pl.ds(base, _BPW)], p1)
    pltpu.sync_copy(aux_ref, aux_v)
    wv = aux_v[pl.ds(0, D)] * (1.0 / (2.0 * NPAIRS))
    bv = aux_v[pl.ds(D, 16)]
    lane = lax.iota(jnp.int32, 16)
    for g in range(_BPW // 16):
        def body(j, acc):
            r = g * 16 + j
            s = p0[r, pl.ds(0, 16)] + p1[r, pl.ds(0, 16)]
            q = p0[r, pl.ds(16, 16)] + p1[r, pl.ds(16, 16)]
            x = (s * s - q) * wv
            z = jnp.sum(x)
            return jnp.where(lane == j, z, acc)

        acc = lax.fori_loop(0, 16, body, jnp.zeros(16, jnp.float32))
        ov[pl.ds(g * 16, 16)] = 1.0 / (1.0 + jnp.exp(-(acc + bv)))
    pltpu.sync_copy(ov, out_ref.at[pl.ds(wid * _BPW, _BPW)])


def kernel(dense_inputs, sparse_inputs, tables, W, b):
    del dense_inputs
    tab = jnp.transpose(tables, (0, 2, 1))
    tail = jnp.pad(tab[:, :, (V // 128) * 128:], ((0, 0), (0, 0), (0, 96)))
    idxT = sparse_inputs.T.reshape(F, B // 128, 128)
    aux = jnp.concatenate([W.reshape(D), jnp.broadcast_to(b, (16,))]
                          ).astype(jnp.float32)
    mesh = plsc.VectorSubcoreMesh(core_axis_name="c", subcore_axis_name="s")
    partials = pl.kernel(
        _phase1_body,
        mesh=mesh,
        compiler_params=pltpu.CompilerParams(
            needs_layout_passes=False, use_tc_tiling_on_sc=True),
        out_type=jax.ShapeDtypeStruct((_NC, B, 128), jnp.float32),
        scratch_types=[
            pltpu.VMEM((2, D, _HCAP2), jnp.float32),
            pltpu.VMEM((2, B // 128, 128), jnp.int32),
            pltpu.VMEM((2, _HCAP + 16), jnp.int32),
            pltpu.VMEM((2, _HCAP + 16), jnp.int32),
            pltpu.VMEM((_NCHK, 32), jnp.int32),
            pltpu.VMEM((2, 32, 128), jnp.float32),
            pltpu.VMEM_SHARED((_ACCR, 128), jnp.float32),
            pltpu.SemaphoreType.DMA,
            pltpu.SemaphoreType.DMA,
            pltpu.SemaphoreType.DMA,
        ],
    )(tab, tail, idxT)
    out = pl.kernel(
        _phase2_body,
        mesh=mesh,
        compiler_params=pltpu.CompilerParams(
            needs_layout_passes=False, use_tc_tiling_on_sc=True),
        out_type=jax.ShapeDtypeStruct((B,), jnp.float32),
        scratch_types=[
            pltpu.VMEM((_BPW, 128), jnp.float32),
            pltpu.VMEM((_BPW, 128), jnp.float32),
            pltpu.VMEM((_BPW,), jnp.float32),
            pltpu.VMEM((2 * 16,), jnp.float32),
        ],
    )(partials, aux)
    return out.reshape(B, 1)
